# Initial kernel scaffold; baseline (speedup 1.0000x reference)
import jax
import jax.numpy as jnp
from jax.experimental import pallas as pl


def kernel(x, edge_index, batch, eps1, W1a, b1a, W1b, b1b, g1, be1, eps2, W2a, b2a, W2b, b2b, g2, be2, eps3, W3a, b3a, W3b, b3b, g3, be3, l1W, l1b, l2W, l2b):
    raise NotImplementedError("write your pallas kernel here")


# speedup vs baseline: 4.6716x; 4.6716x over previous
import functools

import jax
import jax.numpy as jnp
from jax import lax
from jax.experimental import pallas as pl
from jax.experimental.pallas import tpu as pltpu
from jax.experimental.pallas import tpu_sc as plsc

_N = 10000
_E = 320000
_D = 128
_H = 128
_OUT = 10
_G = 128

_NC = 2
_NS = 16
_CHUNK = 128
_CH_PER_TEC = 79
_E_PER_TEC = _CHUNK * _CH_PER_TEC
_E_PER_SC = _E_PER_TEC * _NS
_E_PAD = _E_PER_SC * _NC
_ACC_ROWS = 10240
_ZROWS = _ACC_ROWS // _NS
_OROWS = _N // _NS


def _sc_agg_body(x_hbm, srcs_hbm, dsts_hbm, zeros_hbm, out_hbm,
                 acc_sh, src_v, dst_v, rows_v, sem):
    c = lax.axis_index("c")
    s = lax.axis_index("s")
    pltpu.sync_copy(zeros_hbm, acc_sh.at[pl.ds(s * _ZROWS, _ZROWS)])
    pltpu.sync_copy(srcs_hbm.at[c, s], src_v)
    pltpu.sync_copy(dsts_hbm.at[c, s], dst_v)
    plsc.subcore_barrier()

    def step(j, carry):
        pltpu.async_copy(x_hbm.at[src_v.at[j]], rows_v, sem).wait()
        pltpu.sync_copy(rows_v, acc_sh.at[dst_v.at[j]], add=True)
        return carry

    lax.fori_loop(0, _CH_PER_TEC, step, 0)
    plsc.subcore_barrier()
    pltpu.sync_copy(acc_sh.at[pl.ds(s * _ZROWS, _ZROWS)],
                    out_hbm.at[c, pl.ds(s * _ZROWS, _ZROWS)])


@functools.cache
def _build_sc_agg():
    return functools.partial(
        pl.kernel,
        out_type=jax.ShapeDtypeStruct((_NC, _ACC_ROWS, _D), jnp.float32),
        mesh=plsc.VectorSubcoreMesh(core_axis_name="c", subcore_axis_name="s",
                                    num_cores=_NC, num_subcores=_NS),
        scratch_types=[
            pltpu.VMEM_SHARED((_ACC_ROWS, _D), jnp.float32),
            pltpu.VMEM((_CH_PER_TEC, _CHUNK), jnp.int32),
            pltpu.VMEM((_CH_PER_TEC, _CHUNK), jnp.int32),
            pltpu.VMEM((_CHUNK, _D), jnp.float32),
            pltpu.SemaphoreType.DMA,
        ],
    )(_sc_agg_body)


def _sc_agg(h, srcs, dsts, zeros):
    return _build_sc_agg()(h, srcs, dsts, zeros)


def _layer_body(x_ref, p0_ref, p1_ref, eps_ref, wa_ref, ba_ref, wb_ref, bb_ref,
                g_ref, be_ref, o_ref):
    h = eps_ref[...] * x_ref[...] + (p0_ref[pl.ds(0, _N), :] + p1_ref[pl.ds(0, _N), :])
    h = jnp.maximum(jnp.dot(h, wa_ref[...], preferred_element_type=jnp.float32)
                    + ba_ref[...], 0.0)
    h = jnp.maximum(jnp.dot(h, wb_ref[...], preferred_element_type=jnp.float32)
                    + bb_ref[...], 0.0)
    mu = jnp.mean(h, axis=0)
    d = h - mu
    var = jnp.mean(d * d, axis=0)
    o_ref[...] = g_ref[...] * d * lax.rsqrt(var + 1e-5) + be_ref[...]


def _layer(x, p0, p1, eps, wa, ba, wb, bb, g, be):
    return pl.pallas_call(
        _layer_body,
        out_shape=jax.ShapeDtypeStruct((_N, _H), jnp.float32),
    )(x, p0, p1, jnp.reshape(1.0 + eps, (1, 1)), wa,
      jnp.reshape(ba, (1, _H)), wb, jnp.reshape(bb, (1, _H)),
      jnp.reshape(g, (1, _H)), jnp.reshape(be, (1, _H)))


def _head_body(h_ref, bt_ref, l1w_ref, l1b_ref, l2w_ref, l2b_ref, o_ref):
    onehot = (bt_ref[...][:, None]
              == lax.broadcasted_iota(jnp.int32, (1, _G), 1)).astype(jnp.float32)
    sums = lax.dot_general(onehot, h_ref[...], (((0,), (0,)), ((), ())),
                           preferred_element_type=jnp.float32)
    cnt = jnp.sum(onehot, axis=0)
    pooled = sums / jnp.maximum(cnt, 1.0)[:, None]
    hh = jnp.maximum(jnp.dot(pooled, l1w_ref[...],
                             preferred_element_type=jnp.float32) + l1b_ref[...], 0.0)
    hh = jnp.dot(hh, l2w_ref[...], preferred_element_type=jnp.float32) + l2b_ref[...]
    m = jnp.max(hh, axis=-1, keepdims=True)
    lse = m + jnp.log(jnp.sum(jnp.exp(hh - m), axis=-1, keepdims=True))
    o_ref[...] = hh - lse


def _head(h, batch, l1w, l1b, l2w, l2b):
    return pl.pallas_call(
        _head_body,
        out_shape=jax.ShapeDtypeStruct((_G, _OUT), jnp.float32),
    )(h, batch, l1w, jnp.reshape(l1b, (1, _H)), l2w, jnp.reshape(l2b, (1, _OUT)))


def kernel(x, edge_index, batch, eps1, W1a, b1a, W1b, b1b, g1, be1,
           eps2, W2a, b2a, W2b, b2b, g2, be2,
           eps3, W3a, b3a, W3b, b3b, g3, be3, l1W, l1b, l2W, l2b):
    src = edge_index[0]
    dst = edge_index[1]
    npad = _E_PAD - _E
    srcs = jnp.reshape(
        jnp.concatenate([src, jnp.zeros((npad,), jnp.int32)]),
        (_NC, _NS, _CH_PER_TEC, _CHUNK))
    dsts = jnp.reshape(
        jnp.concatenate([dst, jnp.full((npad,), _N, jnp.int32)]),
        (_NC, _NS, _CH_PER_TEC, _CHUNK))
    zeros = jnp.zeros((_ZROWS, _D), jnp.float32)

    h = x
    for eps, wa, ba, wb, bb, g, be in (
            (eps1, W1a, b1a, W1b, b1b, g1, be1),
            (eps2, W2a, b2a, W2b, b2b, g2, be2),
            (eps3, W3a, b3a, W3b, b3b, g3, be3)):
        parts = _sc_agg(h, srcs, dsts, zeros)
        h = _layer(h, parts[0], parts[1], eps, wa, ba, wb, bb, g, be)
    return _head(h, batch, l1W, l1b, l2W, l2b)
